# Initial kernel scaffold; baseline (speedup 1.0000x reference)
import jax
import jax.numpy as jnp
from jax.experimental import pallas as pl


def kernel(x, edge_index, W_emb, b_emb, conv_w, conv_b, bn_g, bn_b, W_out, b_out):
    raise NotImplementedError("write your pallas kernel here")


# speedup vs baseline: 10.0085x; 10.0085x over previous
import functools

import jax
import jax.numpy as jnp
from jax import lax
from jax.experimental import pallas as pl
from jax.experimental.pallas import tpu as pltpu
from jax.experimental.pallas import tpu_sc as plsc

NC = 2
NS = 16
CHUNK = 80
N_PAD = 10240
H = 128
DEG_W = 16
EPS = 1e-5

_MESH = plsc.VectorSubcoreMesh(
    core_axis_name="c", subcore_axis_name="s", num_cores=NC, num_subcores=NS
)


def _degree_body(dst_hbm, ones_hbm, zeros_hbm, out_hbm, dstv, ones_v, acc_sh):
    c = lax.axis_index("c")
    s = lax.axis_index("s")
    rpt = N_PAD // NS
    base = s * rpt
    pltpu.sync_copy(zeros_hbm.at[pl.ds(base, rpt), :],
                    acc_sh.at[pl.ds(base, rpt), :])
    pltpu.sync_copy(ones_hbm, ones_v)
    plsc.subcore_barrier()

    e_total = dst_hbm.shape[0]
    ept = e_total // (NC * NS)
    tile_base = (c * NS + s) * ept
    nchunks = ept // CHUNK

    def body(j, carry):
        eb = tile_base + j * CHUNK
        pltpu.sync_copy(dst_hbm.at[pl.ds(eb, CHUNK)], dstv)
        pltpu.sync_copy(ones_v, acc_sh.at[dstv], add=True)
        return carry

    lax.fori_loop(0, nchunks, body, 0)
    plsc.subcore_barrier()
    pltpu.sync_copy(acc_sh.at[pl.ds(base, rpt), :],
                    out_hbm.at[c, pl.ds(base, rpt), :])


def _degree_call(dst):
    e = dst.shape[0]
    kern = functools.partial(
        pl.kernel,
        out_type=jax.ShapeDtypeStruct((NC, N_PAD, DEG_W), jnp.float32),
        mesh=_MESH,
        scratch_types=[
            pltpu.VMEM((CHUNK,), jnp.int32),
            pltpu.VMEM((CHUNK, DEG_W), jnp.float32),
            pltpu.VMEM_SHARED((N_PAD, DEG_W), jnp.float32),
        ],
    )(_degree_body)
    ones = jnp.ones((CHUNK, DEG_W), jnp.float32)
    zeros = jnp.zeros((N_PAD, DEG_W), jnp.float32)
    return kern(dst, ones, zeros)


def _propagate_body(z_hbm, src_hbm, dst_hbm, out_hbm, src_v, dst_v, rows_v,
                    acc_sh, sem):
    c = lax.axis_index("c")
    s = lax.axis_index("s")
    rpt = N_PAD // NS
    base = s * rpt
    pltpu.sync_copy(z_hbm.at[pl.ds(base, rpt), :],
                    acc_sh.at[pl.ds(base, rpt), :])
    plsc.subcore_barrier()

    e_total = src_hbm.shape[0]
    ept = e_total // (NC * NS)
    tile_base = (c * NS + s) * ept
    nchunks = ept // CHUNK

    def body(j, carry):
        eb = tile_base + j * CHUNK
        pltpu.sync_copy(src_hbm.at[pl.ds(eb, CHUNK)], src_v)
        pltpu.async_copy(z_hbm.at[src_v], rows_v, sem).wait()
        pltpu.sync_copy(dst_hbm.at[pl.ds(eb, CHUNK)], dst_v)
        pltpu.sync_copy(rows_v, acc_sh.at[dst_v], add=True)
        return carry

    lax.fori_loop(0, nchunks, body, 0)
    plsc.subcore_barrier()
    pltpu.sync_copy(acc_sh.at[pl.ds(base, rpt), :],
                    out_hbm.at[c, pl.ds(base, rpt), :])


def _propagate_call(z, src, dst):
    kern = functools.partial(
        pl.kernel,
        out_type=jax.ShapeDtypeStruct((NC, N_PAD, H), jnp.float32),
        mesh=_MESH,
        scratch_types=[
            pltpu.VMEM((CHUNK,), jnp.int32),
            pltpu.VMEM((CHUNK,), jnp.int32),
            pltpu.VMEM((CHUNK, H), jnp.float32),
            pltpu.VMEM_SHARED((N_PAD, H), jnp.float32),
            pltpu.SemaphoreType.DMA,
        ],
    )(_propagate_body)
    return kern(z, src, dst)


def _dinv_body(parts_ref, o_ref):
    deg = parts_ref[0, :, 0:1] + parts_ref[1, :, 0:1] + 1.0
    o_ref[...] = lax.rsqrt(deg)


def _dinv_call(parts):
    return pl.pallas_call(
        _dinv_body,
        out_shape=jax.ShapeDtypeStruct((N_PAD, 1), jnp.float32),
    )(parts)


def _emb_mm_body(n, x_ref, wemb_ref, bemb_ref, w0_ref, dinv_ref, o_ref):
    h = jnp.dot(x_ref[...], wemb_ref[...], preferred_element_type=jnp.float32)
    h = h + bemb_ref[...]
    xw = jnp.dot(h, w0_ref[...], preferred_element_type=jnp.float32)
    o_ref[0:n, :] = xw * dinv_ref[0:n, :]
    o_ref[n:, :] = jnp.zeros((N_PAD - n, o_ref.shape[1]), jnp.float32)


def _emb_mm_call(x, w_emb, b_emb, w0, dinv):
    n = x.shape[0]
    return pl.pallas_call(
        functools.partial(_emb_mm_body, n),
        out_shape=jax.ShapeDtypeStruct((N_PAD, H), jnp.float32),
    )(x, w_emb, b_emb.reshape(1, -1), w0, dinv)


def _mm_scale_body(n, h_ref, w_ref, dinv_ref, o_ref):
    xw = jnp.dot(h_ref[...], w_ref[...], preferred_element_type=jnp.float32)
    o_ref[0:n, :] = xw * dinv_ref[0:n, :]
    o_ref[n:, :] = jnp.zeros((N_PAD - n, o_ref.shape[1]), jnp.float32)


def _mm_scale_call(h, w, dinv):
    n = h.shape[0]
    return pl.pallas_call(
        functools.partial(_mm_scale_body, n),
        out_shape=jax.ShapeDtypeStruct((N_PAD, H), jnp.float32),
    )(h, w, dinv)


def _combine_bn_body(n, parts_ref, z_ref, dinv_ref, cb_ref, g_ref, b_ref,
                     o_ref):
    t = parts_ref[0, 0:n, :] + parts_ref[1, 0:n, :] - z_ref[0:n, :]
    t = t * dinv_ref[0:n, :] + cb_ref[...]
    mean = jnp.mean(t, axis=0, keepdims=True)
    d = t - mean
    var = jnp.mean(d * d, axis=0, keepdims=True)
    o_ref[...] = jnp.maximum(
        d * lax.rsqrt(var + EPS) * g_ref[...] + b_ref[...], 0.0)


def _combine_bn_call(n, parts, z, dinv, cb, g, b):
    return pl.pallas_call(
        functools.partial(_combine_bn_body, n),
        out_shape=jax.ShapeDtypeStruct((n, H), jnp.float32),
    )(parts, z, dinv, cb.reshape(1, -1), g.reshape(1, -1), b.reshape(1, -1))


def _pool_out_body(h_ref, wout_ref, bout_ref, o_ref):
    pooled = jnp.mean(h_ref[...], axis=0, keepdims=True)
    o_ref[...] = jnp.dot(pooled, wout_ref[...],
                         preferred_element_type=jnp.float32) + bout_ref[...]


def _pool_out_call(h, w_out, b_out):
    o = w_out.shape[1]
    return pl.pallas_call(
        _pool_out_body,
        out_shape=jax.ShapeDtypeStruct((1, o), jnp.float32),
    )(h, w_out, b_out.reshape(1, -1))


def kernel(x, edge_index, W_emb, b_emb, conv_w, conv_b, bn_g, bn_b, W_out,
           b_out):
    n = x.shape[0]
    e = edge_index.shape[1]
    layers = conv_w.shape[0]

    src = edge_index[0]
    dst = edge_index[1]
    step = NC * NS * CHUNK
    e_pad = ((e + step - 1) // step) * step
    if e_pad != e:
        pad = jnp.full((e_pad - e,), N_PAD - 1, dtype=src.dtype)
        src = jnp.concatenate([src, pad])
        dst = jnp.concatenate([dst, pad])

    deg_parts = _degree_call(dst)
    dinv = _dinv_call(deg_parts)

    z = _emb_mm_call(x, W_emb, b_emb, conv_w[0], dinv)
    h = None
    for i in range(layers):
        parts = _propagate_call(z, src, dst)
        h = _combine_bn_call(n, parts, z, dinv, conv_b[i], bn_g[i], bn_b[i])
        if i + 1 < layers:
            z = _mm_scale_call(h, conv_w[i + 1], dinv)

    return _pool_out_call(h, W_out, b_out)
